# baseline (speedup 1.0000x reference)
import jax
import jax.numpy as jnp
from jax import lax
from jax.experimental import pallas as pl
from jax.experimental.pallas import tpu as pltpu
from jax.experimental.pallas import tpu_sc as plsc

_ROW = 6
_N = 24
_L = 16


def _sc_body(z_hbm, bmask_hbm, out_hbm, zv, bv, zsem, bsem):
    zcopy = pltpu.async_copy(z_hbm, zv.at[pl.ds(0, _N)], zsem)
    bcopy = pltpu.async_copy(bmask_hbm, bv.at[pl.ds(0, 2)], bsem)
    bcopy.wait()
    lanes = lax.iota(jnp.int32, _L)
    bvec = bv[...]
    b0 = jnp.sum(jnp.where(lanes == 0, bvec, 0))
    b1 = jnp.sum(jnp.where(lanes == 1, bvec, 0))
    zcopy.wait()
    for chunk in range(2):
        flat = lanes + chunk * _L
        row = lax.div(flat, _ROW)
        member = jnp.logical_or(row == b0, row == b1)
        vals = zv[pl.ds(chunk * _L, _L)]
        zv[pl.ds(chunk * _L, _L)] = jnp.where(member, vals - 0.1, vals)
    pltpu.sync_copy(zv.at[pl.ds(0, _N)], out_hbm)


def kernel(z, bmask):
    zf = jnp.reshape(z, (_N,))
    mesh = plsc.VectorSubcoreMesh(
        core_axis_name="c", subcore_axis_name="s",
        num_cores=1, num_subcores=1)
    run = pl.kernel(
        _sc_body,
        mesh=mesh,
        out_type=jax.ShapeDtypeStruct((_N,), jnp.float32),
        compiler_params=pltpu.CompilerParams(needs_layout_passes=False),
        scratch_types=[
            pltpu.VMEM((2 * _L,), jnp.float32),
            pltpu.VMEM((_L,), jnp.int32),
            pltpu.SemaphoreType.DMA,
            pltpu.SemaphoreType.DMA,
        ],
    )
    return jnp.reshape(run(zf, bmask), z.shape)


# speedup vs baseline: 1.0106x; 1.0106x over previous
import jax
import jax.numpy as jnp
from jax import lax
from jax.experimental import pallas as pl
from jax.experimental.pallas import tpu as pltpu
from jax.experimental.pallas import tpu_sc as plsc

_ROW = 6
_N = 24
_L = 16


def _sc_body(z_hbm, bmask_hbm, out_hbm, zv, bv, zsem, bsem):
    zcopy = pltpu.async_copy(z_hbm, zv.at[pl.ds(0, _N)], zsem)
    bcopy = pltpu.async_copy(bmask_hbm, bv.at[pl.ds(0, 2)], bsem)
    bcopy.wait()
    lanes = lax.iota(jnp.int32, _L)
    b0 = plsc.load_gather(bv, [jnp.zeros((_L,), jnp.int32)])
    b1 = plsc.load_gather(bv, [jnp.ones((_L,), jnp.int32)])
    zcopy.wait()
    for chunk in range(2):
        flat = lanes + chunk * _L
        row = lax.div(flat, _ROW)
        member = jnp.logical_or(row == b0, row == b1)
        vals = zv[pl.ds(chunk * _L, _L)]
        zv[pl.ds(chunk * _L, _L)] = jnp.where(member, vals - 0.1, vals)
    pltpu.sync_copy(zv.at[pl.ds(0, _N)], out_hbm)


def kernel(z, bmask):
    zf = jnp.reshape(z, (_N,))
    mesh = plsc.VectorSubcoreMesh(
        core_axis_name="c", subcore_axis_name="s",
        num_cores=1, num_subcores=1)
    run = pl.kernel(
        _sc_body,
        mesh=mesh,
        out_type=jax.ShapeDtypeStruct((_N,), jnp.float32),
        compiler_params=pltpu.CompilerParams(needs_layout_passes=False),
        scratch_types=[
            pltpu.VMEM((2 * _L,), jnp.float32),
            pltpu.VMEM((_L,), jnp.int32),
            pltpu.SemaphoreType.DMA,
            pltpu.SemaphoreType.DMA,
        ],
    )
    return jnp.reshape(run(zf, bmask), z.shape)
